# baseline (speedup 1.0000x reference)
import jax
import jax.numpy as jnp
from jax.experimental import pallas as pl
from jax.experimental.pallas import tpu as pltpu

N = 10000
D_IN = 256
D_OUT = 256

BM = 200
NB = 4
STEPS = N // BM


def _fused_kernel(x_hbm, w_ref, adj_hbm, out_ref,
                  xv, h, bufs, sems, xsem):
    i = pl.program_id(0)

    @pl.when(i == 0)
    def _prologue():
        pltpu.make_async_copy(x_hbm, xv, xsem).start()
        for j in range(NB):
            pltpu.make_async_copy(
                adj_hbm.at[pl.ds(j * BM, BM), :], bufs.at[j],
                sems.at[j]).start()
        pltpu.make_async_copy(x_hbm, xv, xsem).wait()
        h[...] = jnp.dot(xv[...], w_ref[...],
                         preferred_element_type=jnp.float32)

    slot = jax.lax.rem(i, NB)
    pltpu.make_async_copy(
        adj_hbm.at[pl.ds(i * BM, BM), :], bufs.at[slot],
        sems.at[slot]).wait()
    acc = jnp.dot(bufs[slot], h[...], preferred_element_type=jnp.float32)
    out_ref[...] = jnp.maximum(acc, 0.0)

    @pl.when(i + NB < STEPS)
    def _refill():
        pltpu.make_async_copy(
            adj_hbm.at[pl.ds((i + NB) * BM, BM), :], bufs.at[slot],
            sems.at[slot]).start()


@jax.jit
def kernel(x, adj, W):
    out = pl.pallas_call(
        _fused_kernel,
        grid=(STEPS,),
        in_specs=[
            pl.BlockSpec(memory_space=pl.ANY),
            pl.BlockSpec((D_IN, D_OUT), lambda i: (0, 0)),
            pl.BlockSpec(memory_space=pl.ANY),
        ],
        out_specs=pl.BlockSpec((BM, D_OUT), lambda i: (i, 0)),
        out_shape=jax.ShapeDtypeStruct((N, D_OUT), jnp.float32),
        scratch_shapes=[
            pltpu.VMEM((N, D_IN), jnp.float32),
            pltpu.VMEM((N, D_OUT), jnp.float32),
            pltpu.VMEM((NB, BM, N), jnp.float32),
            pltpu.SemaphoreType.DMA((NB,)),
            pltpu.SemaphoreType.DMA,
        ],
    )(x, W, adj)

    return (out, adj)


# speedup vs baseline: 1.0066x; 1.0066x over previous
import jax
import jax.numpy as jnp
from jax.experimental import pallas as pl
from jax.experimental.pallas import tpu as pltpu

N = 10000
D_IN = 256
D_OUT = 256

BM = 400


def _fused_kernel(x_ref, w_ref, adj_ref, out_ref, h_scratch):
    @pl.when(pl.program_id(0) == 0)
    def _compute_hidden():
        h_scratch[...] = jnp.dot(x_ref[...], w_ref[...],
                                 preferred_element_type=jnp.float32)

    acc = jnp.dot(adj_ref[...], h_scratch[...],
                  preferred_element_type=jnp.float32)
    out_ref[...] = jnp.maximum(acc, 0.0)


@jax.jit
def kernel(x, adj, W):
    out = pl.pallas_call(
        _fused_kernel,
        grid=(N // BM,),
        in_specs=[
            pl.BlockSpec((N, D_IN), lambda i: (0, 0),
                         pipeline_mode=pl.Buffered(buffer_count=1)),
            pl.BlockSpec((D_IN, D_OUT), lambda i: (0, 0),
                         pipeline_mode=pl.Buffered(buffer_count=1)),
            pl.BlockSpec((BM, N), lambda i: (i, 0)),
        ],
        out_specs=pl.BlockSpec((BM, D_OUT), lambda i: (i, 0)),
        out_shape=jax.ShapeDtypeStruct((N, D_OUT), jnp.float32),
        scratch_shapes=[pltpu.VMEM((N, D_OUT), jnp.float32)],
    )(x, W, adj)

    return (out, adj)
